# baseline (speedup 1.0000x reference)
import functools

import jax
import jax.numpy as jnp
from jax import lax
from jax.experimental import pallas as pl
from jax.experimental.pallas import tpu as pltpu
from jax.experimental.pallas import tpu_sc as plsc

_LANES = 16


@functools.lru_cache(maxsize=None)
def _build_sc_kernel(B, D, NS, NQ):
    info = plsc.get_sparse_core_info()
    nc, ns, lanes = info.num_cores, info.num_subcores, info.num_lanes
    nw = nc * ns
    assert lanes == _LANES
    assert B % (8 * nw) == 0 and D == _LANES
    bpw = B // nw
    groups = bpw // lanes

    mesh = plsc.VectorSubcoreMesh(core_axis_name="c", subcore_axis_name="s")

    @functools.partial(
        pl.kernel,
        out_type=jax.ShapeDtypeStruct((B,), jnp.float32),
        mesh=mesh,
        compiler_params=pltpu.CompilerParams(
            needs_layout_passes=False, use_tc_tiling_on_sc=False),
        scratch_types=[
            pltpu.VMEM((bpw,), jnp.int32),
            pltpu.VMEM((bpw,), jnp.int32),
            pltpu.VMEM((bpw, D), jnp.float32),
            pltpu.VMEM((bpw, D), jnp.float32),
            pltpu.VMEM((bpw,), jnp.float32),
            pltpu.VMEM((bpw,), jnp.float32),
            pltpu.SemaphoreType.DMA,
        ],
    )
    def sc_kernel(sid_hbm, qid_hbm, theta_hbm, alpha_hbm, beta_hbm, out_hbm,
                  sidx_v, qidx_v, th_v, al_v, be_v, out_v, sem):
        wid = lax.axis_index("s") * nc + lax.axis_index("c")
        base = wid * bpw

        pltpu.sync_copy(sid_hbm.at[pl.ds(base, bpw)], sidx_v)
        pltpu.sync_copy(qid_hbm.at[pl.ds(base, bpw)], qidx_v)

        cp_th = pltpu.async_copy(theta_hbm.at[sidx_v], th_v, sem)
        cp_al = pltpu.async_copy(alpha_hbm.at[qidx_v], al_v, sem)
        cp_be = pltpu.async_copy(beta_hbm.at[qidx_v], be_v, sem)
        cp_th.wait()
        cp_al.wait()
        cp_be.wait()

        iota = lax.iota(jnp.int32, lanes)

        def group(g, carry):
            rows = g * lanes + iota
            acc = be_v[pl.ds(g * lanes, lanes)]
            for d in range(D):
                col = jnp.full((lanes,), d, jnp.int32)
                tv = plsc.load_gather(th_v, [rows, col])
                av = plsc.load_gather(al_v, [rows, col])
                acc = acc + tv * av
            out_v[pl.ds(g * lanes, lanes)] = 1.0 / (1.0 + jnp.exp(-acc))
            return carry

        lax.fori_loop(0, groups, group, 0)

        pltpu.sync_copy(out_v, out_hbm.at[pl.ds(base, bpw)])

    return sc_kernel


def kernel(student_ids, question_ids, theta, alpha, beta):
    B = student_ids.shape[0]
    NS, D = theta.shape
    NQ = alpha.shape[0]
    fn = _build_sc_kernel(B, D, NS, NQ)
    out = fn(student_ids.astype(jnp.int32), question_ids.astype(jnp.int32),
             theta, alpha, beta.reshape(-1))
    return out.reshape(B, 1)
